# baseline (speedup 1.0000x reference)
import functools

import jax
import jax.numpy as jnp
from jax import lax
from jax.experimental import pallas as pl
from jax.experimental.pallas import tpu as pltpu
from jax.experimental.pallas import tpu_sc as plsc

NUM_TOKENS = 16384
SEQ = 50
DIM = 32
NUM_ROWS = 1000000

_info = plsc.get_sparse_core_info()
NC, NS = _info.num_cores, _info.num_subcores
NW = NC * NS
TW = NUM_TOKENS // NW
TT = TW // 128
DT = DIM // 8


def _emb_body(
    idx_hbm, table_hbm, out_hbm, idx_v, rows_v, out_v,
    si, sg0, sg1, sg2, sg3, ss0, ss1,
):
    wid = lax.axis_index("s") * NC + lax.axis_index("c")
    tt0 = wid * TT
    sg = (sg0, sg1, sg2, sg3)
    ss = (ss0, ss1)
    iota16 = lax.iota(jnp.int32, 16)
    col_idx = [jnp.full((16,), d, dtype=jnp.int32) for d in range(DIM)]

    for s in range(SEQ):
        for k in range(TT):
            pltpu.async_copy(
                idx_hbm.at[s // 8, tt0 + k, s % 8, :],
                idx_v.at[s, pl.ds(k * 128, 128)],
                si,
            )
    for _ in range(SEQ * TT):
        pltpu.make_async_copy(
            idx_hbm.at[0, tt0, 0, :], idx_v.at[0, pl.ds(0, 128)], si
        ).wait()
    zero16 = jnp.zeros((16,), jnp.int32)
    for r in range(SEQ, SEQ + 3):
        for k in range(TW // 16):
            idx_v[r, pl.ds(k * 16, 16)] = zero16

    def start_gather(s, slot):
        pltpu.async_copy(table_hbm.at[idx_v.at[s]], rows_v.at[slot], sg[slot])

    def wait_gather(slot):
        pltpu.make_async_copy(
            table_hbm.at[idx_v.at[0]], rows_v.at[slot], sg[slot]
        ).wait()

    def start_store(s, slot):
        pltpu.async_copy(
            out_v.at[slot], out_hbm.at[s, :, pl.ds(tt0, TT), :, :], ss[slot]
        )

    def wait_store(slot):
        pltpu.make_async_copy(
            out_v.at[0], out_hbm.at[0, :, pl.ds(tt0, TT), :, :], ss[slot]
        ).wait()

    def transpose(gslot, oslot):
        @plsc.parallel_loop(0, TW // 16, step=1, unroll=2)
        def jbody(j):
            jo = j // 8
            ji = (j % 8) * 16
            row_idx = iota16 + j * 16
            vals = [
                plsc.load_gather(rows_v.at[gslot], [row_idx, col_idx[d]])
                for d in range(DIM)
            ]
            for d in range(DIM):
                out_v[oslot, d // 8, jo, d % 8, pl.ds(ji, 16)] = vals[d]

    def do_iter(s, gslot, oslot, wait_prev_store):
        start_gather(s + 3, (gslot + 3) % 4)
        wait_gather(gslot)
        if wait_prev_store:
            wait_store(oslot)
        transpose(gslot, oslot)
        start_store(s, oslot)

    start_gather(0, 0)
    start_gather(1, 1)
    start_gather(2, 2)
    do_iter(0, 0, 0, False)
    do_iter(1, 1, 1, False)

    def body(i, carry):
        s = 2 + 4 * i
        do_iter(s, 2, 0, True)
        do_iter(s + 1, 3, 1, True)
        do_iter(s + 2, 0, 0, True)
        do_iter(s + 3, 1, 1, True)
        return carry

    lax.fori_loop(0, (SEQ - 2) // 4, body, 0)

    wait_gather(SEQ % 4)
    wait_gather((SEQ + 1) % 4)
    wait_gather((SEQ + 2) % 4)
    wait_store(0)
    wait_store(1)


_emb = functools.partial(
    pl.kernel,
    mesh=plsc.VectorSubcoreMesh(core_axis_name="c", subcore_axis_name="s"),
    out_type=jax.ShapeDtypeStruct(
        (SEQ, DT, NUM_TOKENS // 128, 8, 128), jnp.float32
    ),
    scratch_types=[
        pltpu.VMEM((SEQ + 3, TW), jnp.int32),
        pltpu.VMEM((4, TW, DIM), jnp.float32),
        pltpu.VMEM((2, DT, TT, 8, 128), jnp.float32),
        pltpu.SemaphoreType.DMA,
        pltpu.SemaphoreType.DMA,
        pltpu.SemaphoreType.DMA,
        pltpu.SemaphoreType.DMA,
        pltpu.SemaphoreType.DMA,
        pltpu.SemaphoreType.DMA,
        pltpu.SemaphoreType.DMA,
    ],
    compiler_params=pltpu.CompilerParams(
        use_tc_tiling_on_sc=False, needs_layout_passes=False
    ),
)(_emb_body)


@jax.jit
def kernel(token_ids, weight):
    tokT = jnp.minimum(token_ids.astype(jnp.int32).T, jnp.int32(NUM_ROWS - 1))
    idx4 = (
        jnp.pad(tokT, ((0, 6), (0, 0)))
        .reshape(7, 8, 128, 128)
        .transpose(0, 2, 1, 3)
    )
    out5 = _emb(idx4, weight)
    out = jnp.transpose(out5, (2, 4, 0, 1, 3)).reshape(NUM_TOKENS, SEQ, DIM)
    return out


# speedup vs baseline: 1.5189x; 1.5189x over previous
import functools

import jax
import jax.numpy as jnp
from jax import lax
from jax.experimental import pallas as pl
from jax.experimental.pallas import tpu as pltpu
from jax.experimental.pallas import tpu_sc as plsc

NUM_TOKENS = 16384
SEQ = 50
DIM = 32
NUM_ROWS = 1000000

_info = plsc.get_sparse_core_info()
NC, NS = _info.num_cores, _info.num_subcores
NW = NC * NS
TW = NUM_TOKENS // NW
TT = TW // 128
DT = DIM // 8


def _emb_body(idx_hbm, table_hbm, out_hbm, idx_v, rows_v, out_v, si, sg0, sg1, ss0, ss1):
    wid = lax.axis_index("s") * NC + lax.axis_index("c")
    tt0 = wid * TT
    sg = (sg0, sg1)
    ss = (ss0, ss1)
    iota16 = lax.iota(jnp.int32, 16)
    col_idx = [jnp.full((16,), d, dtype=jnp.int32) for d in range(DIM)]

    for s in range(SEQ):
        for k in range(TT):
            pltpu.async_copy(
                idx_hbm.at[s // 8, tt0 + k, s % 8, :],
                idx_v.at[s, pl.ds(k * 128, 128)],
                si,
            )
    for _ in range(SEQ * TT):
        pltpu.make_async_copy(
            idx_hbm.at[0, tt0, 0, :], idx_v.at[0, pl.ds(0, 128)], si
        ).wait()

    def start_gather(s, slot):
        pltpu.async_copy(table_hbm.at[idx_v.at[s]], rows_v.at[slot], sg[slot])

    def wait_gather(slot):
        pltpu.make_async_copy(
            table_hbm.at[idx_v.at[0]], rows_v.at[slot], sg[slot]
        ).wait()

    def start_store(s, slot):
        pltpu.async_copy(
            out_v.at[slot], out_hbm.at[s, :, pl.ds(tt0, TT), :, :], ss[slot]
        )

    def wait_store(slot):
        pltpu.make_async_copy(
            out_v.at[0], out_hbm.at[0, :, pl.ds(tt0, TT), :, :], ss[slot]
        ).wait()

    def transpose(slot):
        @plsc.parallel_loop(0, TW // 16, step=1, unroll=2)
        def jbody(j):
            jo = j // 8
            ji = (j % 8) * 16
            row_idx = iota16 + j * 16
            vals = [
                plsc.load_gather(rows_v.at[slot], [row_idx, col_idx[d]])
                for d in range(DIM)
            ]
            for d in range(DIM):
                out_v[slot, d // 8, jo, d % 8, pl.ds(ji, 16)] = vals[d]

    def do_iter(s, slot, wait_prev_store, next_gather):
        if next_gather:
            start_gather(s + 1, 1 - slot)
        wait_gather(slot)
        if wait_prev_store:
            wait_store(slot)
        transpose(slot)
        start_store(s, slot)

    start_gather(0, 0)
    do_iter(0, 0, False, True)
    do_iter(1, 1, False, True)

    def body(i, carry):
        do_iter(2 * i, 0, True, True)
        do_iter(2 * i + 1, 1, True, True)
        return carry

    lax.fori_loop(1, (SEQ - 2) // 2, body, 0)

    do_iter(SEQ - 2, 0, True, True)
    do_iter(SEQ - 1, 1, True, False)
    wait_store(0)
    wait_store(1)


_emb = functools.partial(
    pl.kernel,
    mesh=plsc.VectorSubcoreMesh(core_axis_name="c", subcore_axis_name="s"),
    out_type=jax.ShapeDtypeStruct(
        (SEQ, DT, NUM_TOKENS // 128, 8, 128), jnp.float32
    ),
    scratch_types=[
        pltpu.VMEM((SEQ, TW), jnp.int32),
        pltpu.VMEM((2, TW, DIM), jnp.float32),
        pltpu.VMEM((2, DT, TT, 8, 128), jnp.float32),
        pltpu.SemaphoreType.DMA,
        pltpu.SemaphoreType.DMA,
        pltpu.SemaphoreType.DMA,
        pltpu.SemaphoreType.DMA,
        pltpu.SemaphoreType.DMA,
    ],
    compiler_params=pltpu.CompilerParams(
        use_tc_tiling_on_sc=False, needs_layout_passes=False
    ),
)(_emb_body)


@jax.jit
def kernel(token_ids, weight):
    tokT = jnp.minimum(token_ids.astype(jnp.int32).T, jnp.int32(NUM_ROWS - 1))
    idx4 = (
        jnp.pad(tokT, ((0, 6), (0, 0)))
        .reshape(7, 8, 128, 128)
        .transpose(0, 2, 1, 3)
    )
    out5 = _emb(idx4, weight)
    out = jnp.transpose(out5, (2, 4, 0, 1, 3)).reshape(NUM_TOKENS, SEQ, DIM)
    return out
